# baseline (speedup 1.0000x reference)
import functools

import jax
import jax.numpy as jnp
from jax import lax
from jax.experimental import pallas as pl
from jax.experimental.pallas import tpu as pltpu
from jax.experimental.pallas import tpu_sc as plsc

N = 10000
E = 320000
D = 128
DO = 64

NW = 32
EPW = 10240
EB = 128
NBLK = EPW // EB
EPAD = NW * EPW
NPAD = 10240
CHUNK = NPAD // 16

_mesh = plsc.VectorSubcoreMesh(core_axis_name="c", subcore_axis_name="s")


NQ = (NBLK - 4) // 4

W0 = 152
W1 = 8
C1OFF = 16 * W0
assert 16 * (W0 + W1) * EB == EPAD


@functools.partial(
    pl.kernel,
    mesh=_mesh,
    out_type=jax.ShapeDtypeStruct((2, NPAD, D), jnp.float32),
    scratch_types=[
        pltpu.VMEM((EB, D), jnp.float32),
        pltpu.VMEM((EB,), jnp.int32),
        pltpu.VMEM((EB,), jnp.int32),
        pltpu.VMEM((EB,), jnp.int32),
        pltpu.VMEM((EB,), jnp.int32),
        pltpu.VMEM_SHARED((NPAD, D), jnp.float32),
        pltpu.SemaphoreType.DMA,
        pltpu.SemaphoreType.DMA,
        pltpu.SemaphoreType.DMA,
        pltpu.SemaphoreType.DMA,
        pltpu.SemaphoreType.DMA,
        pltpu.SemaphoreType.DMA,
    ],
)
def _deg_kernel(dst_hbm, ones_hbm, out, rows_v,
                idxd0, idxd1, idxd2, idxd3, acc_sh,
                ss0, ss1, si0, si1, si2, si3):
    cid = lax.axis_index("c")
    sid = lax.axis_index("s")
    wid = sid * 2 + cid
    idxd = [idxd0, idxd1, idxd2, idxd3]
    ss = [ss0, ss1]
    si = [si0, si1, si2, si3]

    for t in range(CHUNK // EB):
        r0 = sid * CHUNK + t * EB
        pltpu.sync_copy(ones_hbm.at[pl.ds(r0, EB)], rows_v)
        pltpu.sync_copy(rows_v, acc_sh.at[pl.ds(r0, EB)])
    plsc.subcore_barrier()

    def base_of(b):
        return wid * EPW + b * EB

    def prefetch(b, I):
        pltpu.async_copy(dst_hbm.at[pl.ds(base_of(b), EB)], idxd[I], si[I])

    def wait_prefetch(b, I):
        pltpu.make_async_copy(
            dst_hbm.at[pl.ds(base_of(b), EB)], idxd[I], si[I]).wait()

    def issue_scatter(B, I):
        pltpu.async_copy(rows_v, acc_sh.at[idxd[I]], ss[B], add=True)

    def wait_scatter(B, I):
        pltpu.make_async_copy(rows_v, acc_sh.at[idxd[I]], ss[B]).wait()

    for j in (0, 1):
        pltpu.sync_copy(dst_hbm.at[pl.ds(base_of(j), EB)], idxd[j])
        issue_scatter(j, j)
        prefetch(j + 2, j + 2)

    @pl.loop(0, NQ)
    def _(q):
        b0 = 2 + 4 * q
        for j in range(4):
            b = b0 + j
            B = j % 2
            I = (2 + j) % 4
            wait_scatter(B, (I + 2) % 4)
            wait_prefetch(b, I)
            issue_scatter(B, I)
            prefetch(b + 2, (I + 2) % 4)

    for j in (0, 1):
        b = NBLK - 2 + j
        B = j
        I = (2 + j) % 4
        wait_scatter(B, (I + 2) % 4)
        wait_prefetch(b, I)
        issue_scatter(B, I)
    wait_scatter(0, 2)
    wait_scatter(1, 3)

    plsc.subcore_barrier()
    for t in range(CHUNK // EB):
        r0 = sid * CHUNK + t * EB
        pltpu.sync_copy(acc_sh.at[pl.ds(r0, EB)], rows_v)
        pltpu.sync_copy(rows_v, out.at[cid, pl.ds(r0, EB)])


@functools.partial(
    pl.kernel,
    mesh=_mesh,
    out_type=jax.ShapeDtypeStruct((2, NPAD, D), jnp.float32),
    scratch_types=[
        pltpu.VMEM((EB, D), jnp.float32),
        pltpu.VMEM((EB, D), jnp.float32),
        pltpu.VMEM((EB,), jnp.int32),
        pltpu.VMEM((EB,), jnp.int32),
        pltpu.VMEM((EB,), jnp.int32),
        pltpu.VMEM((EB,), jnp.int32),
        pltpu.VMEM((EB,), jnp.int32),
        pltpu.VMEM((EB,), jnp.int32),
        pltpu.VMEM_SHARED((NPAD, D), jnp.float32),
        pltpu.SemaphoreType.DMA,
        pltpu.SemaphoreType.DMA,
        pltpu.SemaphoreType.DMA,
        pltpu.SemaphoreType.DMA,
        pltpu.SemaphoreType.DMA,
        pltpu.SemaphoreType.DMA,
        pltpu.SemaphoreType.DMA,
        pltpu.SemaphoreType.DMA,
    ],
)
def _scatter_kernel(g_hbm, src_hbm, dst_hbm, out,
                    rows0, rows1, idxs0, idxs1, idxd0, idxd1, idxd2, idxd3,
                    acc_sh, sg0, sg1, ss0, ss1, si0, si1, si2, si3):
    cid = lax.axis_index("c")
    sid = lax.axis_index("s")
    rows = [rows0, rows1]
    idxs = [idxs0, idxs1]
    idxd = [idxd0, idxd1, idxd2, idxd3]
    sg = [sg0, sg1]
    ss = [ss0, ss1]
    si = [si0, si1, si2, si3]

    for t in range(CHUNK // EB):
        r0 = sid * CHUNK + t * EB
        pltpu.sync_copy(g_hbm.at[pl.ds(r0, EB)], rows0)
        pltpu.sync_copy(rows0, acc_sh.at[pl.ds(r0, EB)])
    plsc.subcore_barrier()

    def run_pipeline(nblk, tbase):
        def base_of(b):
            return tbase + b * EB

        def prefetch(b, B, I):
            pltpu.async_copy(src_hbm.at[pl.ds(base_of(b), EB)], idxs[B], si[I])
            pltpu.async_copy(dst_hbm.at[pl.ds(base_of(b), EB)], idxd[I], si[I])

        def wait_prefetch(b, B, I):
            pltpu.make_async_copy(
                src_hbm.at[pl.ds(base_of(b), EB)], idxs[B], si[I]).wait()
            pltpu.make_async_copy(
                dst_hbm.at[pl.ds(base_of(b), EB)], idxd[I], si[I]).wait()

        def wait_scatter(B, I):
            pltpu.make_async_copy(rows[B], acc_sh.at[idxd[I]], ss[B]).wait()

        def full_slot(b, B, I, do_prefetch=True):
            wait_prefetch(b, B, I)
            h = pltpu.async_copy(g_hbm.at[idxs[B]], rows[B], sg[B])
            h.wait()
            pltpu.async_copy(rows[B], acc_sh.at[idxd[I]], ss[B], add=True)
            if do_prefetch:
                prefetch(b + 2, B, (I + 2) % 4)

        for j in (0, 1):
            pltpu.sync_copy(src_hbm.at[pl.ds(base_of(j), EB)], idxs[j])
            pltpu.sync_copy(dst_hbm.at[pl.ds(base_of(j), EB)], idxd[j])
        for j in (0, 1):
            h = pltpu.async_copy(g_hbm.at[idxs[j]], rows[j], sg[j])
            h.wait()
            pltpu.async_copy(rows[j], acc_sh.at[idxd[j]], ss[j], add=True)
            prefetch(j + 2, j, j + 2)

        @pl.loop(0, (nblk - 4) // 4)
        def _(q):
            b0 = 2 + 4 * q
            for j in range(4):
                B = j % 2
                I = (2 + j) % 4
                wait_scatter(B, (I + 2) % 4)
                full_slot(b0 + j, B, I)

        for j in (0, 1):
            B = j
            I = (2 + j) % 4
            wait_scatter(B, (I + 2) % 4)
            full_slot(nblk - 2 + j, B, I, do_prefetch=False)
        wait_scatter(0, 2)
        wait_scatter(1, 3)

    @pl.when(cid == 0)
    def _():
        run_pipeline(W0, (sid * W0) * EB)

    if W1 > 0:
        @pl.when(cid == 1)
        def _():
            run_pipeline(W1, (C1OFF + sid * W1) * EB)

    plsc.subcore_barrier()
    for t in range(CHUNK // EB):
        r0 = sid * CHUNK + t * EB
        pltpu.sync_copy(acc_sh.at[pl.ds(r0, EB)], rows0)
        pltpu.sync_copy(rows0, out.at[cid, pl.ds(r0, EB)])


_RB = 2000
_G = (5,)


def _rows(w):
    return pl.BlockSpec((_RB, w), lambda i: (i, 0))


def _const(shape):
    return pl.BlockSpec(shape, lambda i: (0, 0))


def _dinv_body(p0, p1, o):
    deg = p0[:, 0:1] + p1[:, 0:1] - 1.0
    o[...] = lax.rsqrt(deg)


def _dinv_call(p0, p1):
    return pl.pallas_call(
        _dinv_body, grid=_G,
        in_specs=[_rows(D), _rows(D)],
        out_specs=_rows(1),
        out_shape=jax.ShapeDtypeStruct((N, 1), jnp.float32),
    )(p0, p1)


def _lin1_body(x, w, dinv, o):
    o[...] = dinv[...] * jnp.dot(x[...], w[...],
                                 preferred_element_type=jnp.float32)


def _lin1_call(x, W1, dinv):
    return pl.pallas_call(
        _lin1_body, grid=_G,
        in_specs=[_rows(D), _const((D, D)), _rows(1)],
        out_specs=_rows(D),
        out_shape=jax.ShapeDtypeStruct((N, D), jnp.float32),
    )(x, W1, dinv)


def _mid_body(p0, p1, g, dinv, b, w, o):
    h = jnp.maximum(dinv[...] * (p0[...] + p1[...] - g[...]) + b[...], 0.0)
    o[...] = dinv[...] * jnp.dot(h, w[...], preferred_element_type=jnp.float32)


def _mid_call(p0, p1, g, dinv, b, W):
    return pl.pallas_call(
        _mid_body, grid=_G,
        in_specs=[_rows(D), _rows(D), _rows(D), _rows(1),
                  _const((1, D)), _const((D, D))],
        out_specs=_rows(D),
        out_shape=jax.ShapeDtypeStruct((N, D), jnp.float32),
    )(p0, p1, g, dinv, b, W)


def _g3_body(p0, p1, g, dinv, b, o):
    h = jnp.maximum(dinv[...] * (p0[...] + p1[...] - g[...]) + b[...], 0.0)
    o[...] = dinv[...] * h


def _g3_call(p0, p1, g, dinv, b):
    return pl.pallas_call(
        _g3_body, grid=_G,
        in_specs=[_rows(D), _rows(D), _rows(D), _rows(1), _const((1, D))],
        out_specs=_rows(D),
        out_shape=jax.ShapeDtypeStruct((N, D), jnp.float32),
    )(p0, p1, g, dinv, b)


def _out_body(p0, p1, g, dinv, wmu, bmu, wls, bls, omu, ols):
    agg = dinv[...] * (p0[...] + p1[...] - g[...])
    omu[...] = jnp.dot(agg, wmu[...],
                       preferred_element_type=jnp.float32) + bmu[...]
    ols[...] = jnp.dot(agg, wls[...],
                       preferred_element_type=jnp.float32) + bls[...]


def _out_call(p0, p1, g, dinv, Wmu, bmu, Wls, bls):
    return pl.pallas_call(
        _out_body, grid=_G,
        in_specs=[_rows(D), _rows(D), _rows(D), _rows(1),
                  _const((D, DO)), _const((1, DO)),
                  _const((D, DO)), _const((1, DO))],
        out_specs=[_rows(DO), _rows(DO)],
        out_shape=[jax.ShapeDtypeStruct((N, DO), jnp.float32),
                   jax.ShapeDtypeStruct((N, DO), jnp.float32)],
    )(p0, p1, g, dinv, Wmu, bmu, Wls, bls)


def _pad_rows(g):
    return jnp.concatenate([g, jnp.zeros((NPAD - N, D), jnp.float32)], axis=0)


def kernel(x, edge_index, W1, b1, W2, b2, W_mu, b_mu, W_ls, b_ls):
    src = edge_index[0]
    dst = edge_index[1]
    pad = EPAD - E
    srcp = jnp.concatenate([src, jnp.zeros((pad,), jnp.int32)])
    dstp = jnp.concatenate([dst, jnp.full((pad,), N, jnp.int32)])

    onesp = jnp.ones((NPAD, D), jnp.float32)
    dd = _deg_kernel(dstp, onesp)
    dinv = _dinv_call(dd[0], dd[1])

    b1r = b1.reshape(1, D)
    b2r = b2.reshape(1, D)
    bmur = b_mu.reshape(1, DO)
    blsr = b_ls.reshape(1, DO)

    g1 = _lin1_call(x, W1, dinv)
    p = _scatter_kernel(_pad_rows(g1), srcp, dstp)
    g2 = _mid_call(p[0], p[1], g1, dinv, b1r, W2)
    p = _scatter_kernel(_pad_rows(g2), srcp, dstp)
    g3 = _g3_call(p[0], p[1], g2, dinv, b2r)
    p = _scatter_kernel(_pad_rows(g3), srcp, dstp)
    mu, ls = _out_call(p[0], p[1], g3, dinv, W_mu, bmur, W_ls, blsr)
    return (mu, ls)


# speedup vs baseline: 1.7124x; 1.7124x over previous
import functools

import jax
import jax.numpy as jnp
from jax import lax
from jax.experimental import pallas as pl
from jax.experimental.pallas import tpu as pltpu
from jax.experimental.pallas import tpu_sc as plsc

N = 10000
E = 320000
D = 128
DO = 64

NW = 32
EPW = 10240
EB = 128
NBLK = EPW // EB
EPAD = NW * EPW
NPAD = 10240
CHUNK = NPAD // 16

_mesh = plsc.VectorSubcoreMesh(core_axis_name="c", subcore_axis_name="s")


NQ = (NBLK - 4) // 4

EBS = 112
W0 = 161
W1 = 18
WT = W0 + W1
C1OFF = 16 * W0
SPAD = 16 * WT * EBS + 2 * EBS
assert SPAD - 2 * EBS >= E


@functools.partial(
    pl.kernel,
    mesh=_mesh,
    out_type=jax.ShapeDtypeStruct((2, NPAD, D), jnp.float32),
    scratch_types=[
        pltpu.VMEM((EB, D), jnp.float32),
        pltpu.VMEM((EB,), jnp.int32),
        pltpu.VMEM((EB,), jnp.int32),
        pltpu.VMEM((EB,), jnp.int32),
        pltpu.VMEM((EB,), jnp.int32),
        pltpu.VMEM_SHARED((NPAD, D), jnp.float32),
        pltpu.SemaphoreType.DMA,
        pltpu.SemaphoreType.DMA,
        pltpu.SemaphoreType.DMA,
        pltpu.SemaphoreType.DMA,
        pltpu.SemaphoreType.DMA,
        pltpu.SemaphoreType.DMA,
    ],
)
def _deg_kernel(dst_hbm, ones_hbm, out, rows_v,
                idxd0, idxd1, idxd2, idxd3, acc_sh,
                ss0, ss1, si0, si1, si2, si3):
    cid = lax.axis_index("c")
    sid = lax.axis_index("s")
    wid = sid * 2 + cid
    idxd = [idxd0, idxd1, idxd2, idxd3]
    ss = [ss0, ss1]
    si = [si0, si1, si2, si3]

    for t in range(CHUNK // EB):
        r0 = sid * CHUNK + t * EB
        pltpu.sync_copy(ones_hbm.at[pl.ds(r0, EB)], rows_v)
        pltpu.sync_copy(rows_v, acc_sh.at[pl.ds(r0, EB)])
    plsc.subcore_barrier()

    def base_of(b):
        return wid * EPW + b * EB

    def prefetch(b, I):
        pltpu.async_copy(dst_hbm.at[pl.ds(base_of(b), EB)], idxd[I], si[I])

    def wait_prefetch(b, I):
        pltpu.make_async_copy(
            dst_hbm.at[pl.ds(base_of(b), EB)], idxd[I], si[I]).wait()

    def issue_scatter(B, I):
        pltpu.async_copy(rows_v, acc_sh.at[idxd[I]], ss[B], add=True)

    def wait_scatter(B, I):
        pltpu.make_async_copy(rows_v, acc_sh.at[idxd[I]], ss[B]).wait()

    for j in (0, 1):
        pltpu.sync_copy(dst_hbm.at[pl.ds(base_of(j), EB)], idxd[j])
        issue_scatter(j, j)
        prefetch(j + 2, j + 2)

    @pl.loop(0, NQ)
    def _(q):
        b0 = 2 + 4 * q
        for j in range(4):
            b = b0 + j
            B = j % 2
            I = (2 + j) % 4
            wait_scatter(B, (I + 2) % 4)
            wait_prefetch(b, I)
            issue_scatter(B, I)
            prefetch(b + 2, (I + 2) % 4)

    for j in (0, 1):
        b = NBLK - 2 + j
        B = j
        I = (2 + j) % 4
        wait_scatter(B, (I + 2) % 4)
        wait_prefetch(b, I)
        issue_scatter(B, I)
    wait_scatter(0, 2)
    wait_scatter(1, 3)

    plsc.subcore_barrier()
    for t in range(CHUNK // EB):
        r0 = sid * CHUNK + t * EB
        pltpu.sync_copy(acc_sh.at[pl.ds(r0, EB)], rows_v)
        pltpu.sync_copy(rows_v, out.at[cid, pl.ds(r0, EB)])


@functools.partial(
    pl.kernel,
    mesh=_mesh,
    out_type=jax.ShapeDtypeStruct((2, NPAD, D), jnp.float32),
    scratch_types=[
        pltpu.VMEM((EBS, D), jnp.float32),
        pltpu.VMEM((EBS, D), jnp.float32),
        pltpu.VMEM((EBS, D), jnp.float32),
        pltpu.VMEM((EBS,), jnp.int32),
        pltpu.VMEM((EBS,), jnp.int32),
        pltpu.VMEM((EBS,), jnp.int32),
        pltpu.VMEM((EBS,), jnp.int32),
        pltpu.VMEM((EBS,), jnp.int32),
        pltpu.VMEM((EBS,), jnp.int32),
        pltpu.VMEM((EBS,), jnp.int32),
        pltpu.VMEM((EBS,), jnp.int32),
        pltpu.VMEM_SHARED((NPAD, D), jnp.float32),
        pltpu.SemaphoreType.DMA,
        pltpu.SemaphoreType.DMA,
        pltpu.SemaphoreType.DMA,
        pltpu.SemaphoreType.DMA,
        pltpu.SemaphoreType.DMA,
        pltpu.SemaphoreType.DMA,
        pltpu.SemaphoreType.DMA,
        pltpu.SemaphoreType.DMA,
        pltpu.SemaphoreType.DMA,
        pltpu.SemaphoreType.DMA,
    ],
)
def _scatter_kernel(g_hbm, src_hbm, dst_hbm, out,
                    rows0, rows1, rows2,
                    idxs0, idxs1, idxs2, idxs3,
                    idxd0, idxd1, idxd2, idxd3,
                    acc_sh,
                    sg0, sg1, sg2, ss0, ss1, ss2, si0, si1, si2, si3):
    cid = lax.axis_index("c")
    sid = lax.axis_index("s")
    rows = [rows0, rows1, rows2]
    idxs = [idxs0, idxs1, idxs2, idxs3]
    idxd = [idxd0, idxd1, idxd2, idxd3]
    sg = [sg0, sg1, sg2]
    ss = [ss0, ss1, ss2]
    si = [si0, si1, si2, si3]

    RC = 80
    for t in range(CHUNK // RC):
        r0 = sid * CHUNK + t * RC
        pltpu.sync_copy(g_hbm.at[pl.ds(r0, RC)], rows0.at[pl.ds(0, RC)])
        pltpu.sync_copy(rows0.at[pl.ds(0, RC)], acc_sh.at[pl.ds(r0, RC)])
    plsc.subcore_barrier()

    def run_pipeline(nblk, tbase):
        def base_of(b):
            return tbase + b * EBS

        def prefetch(b):
            I = b % 4
            pltpu.async_copy(src_hbm.at[pl.ds(base_of(b), EBS)], idxs[I], si[I])
            pltpu.async_copy(dst_hbm.at[pl.ds(base_of(b), EBS)], idxd[I], si[I])

        def wait_prefetch(b):
            I = b % 4
            pltpu.make_async_copy(
                src_hbm.at[pl.ds(base_of(b), EBS)], idxs[I], si[I]).wait()
            pltpu.make_async_copy(
                dst_hbm.at[pl.ds(base_of(b), EBS)], idxd[I], si[I]).wait()

        def issue_gather(b):
            pltpu.async_copy(g_hbm.at[idxs[b % 4]], rows[b % 3], sg[b % 3])

        def wait_gather(b):
            pltpu.make_async_copy(
                g_hbm.at[idxs[b % 4]], rows[b % 3], sg[b % 3]).wait()

        def issue_scatter(b):
            pltpu.async_copy(rows[b % 3], acc_sh.at[idxd[b % 4]],
                             ss[b % 3], add=True)

        def wait_scatter(b):
            pltpu.make_async_copy(rows[b % 3], acc_sh.at[idxd[b % 4]],
                                  ss[b % 3]).wait()

        def slot(b, steady):
            if steady:
                wait_scatter(b - 3)
            wait_prefetch(b)
            issue_gather(b)
            wait_gather(b - 1)
            issue_scatter(b - 1)
            if b + 1 < nblk:
                prefetch(b + 1)

        pltpu.sync_copy(src_hbm.at[pl.ds(base_of(0), EBS)], idxs[0])
        pltpu.sync_copy(dst_hbm.at[pl.ds(base_of(0), EBS)], idxd[0])
        issue_gather(0)
        prefetch(1)
        for b in (1, 2):
            slot(b, steady=False)
        slot(3, steady=True)

        m12 = (nblk - 4) // 12
        if m12 > 0 and (nblk - 4) % 12 == 0:
            m12 -= 1

        if m12 > 0:
            @pl.loop(0, m12)
            def _(q):
                b0 = 4 + 12 * q
                for j in range(12):
                    bb = b0 + j
                    if True:
                        I3 = (4 + j) % 3
                        I4 = (4 + j) % 4
                        pltpu.make_async_copy(
                            rows[I3],
                            acc_sh.at[idxd[(I4 + 1) % 4]],
                            ss[I3]).wait()
                        pltpu.make_async_copy(
                            src_hbm.at[pl.ds(base_of(bb), EBS)],
                            idxs[I4], si[I4]).wait()
                        pltpu.make_async_copy(
                            dst_hbm.at[pl.ds(base_of(bb), EBS)],
                            idxd[I4], si[I4]).wait()
                        pltpu.async_copy(g_hbm.at[idxs[I4]],
                                         rows[I3], sg[I3])
                        pltpu.make_async_copy(
                            g_hbm.at[idxs[(I4 + 3) % 4]],
                            rows[(I3 + 2) % 3],
                            sg[(I3 + 2) % 3]).wait()
                        pltpu.async_copy(rows[(I3 + 2) % 3],
                                         acc_sh.at[idxd[(I4 + 3) % 4]],
                                         ss[(I3 + 2) % 3], add=True)
                        pltpu.async_copy(
                            src_hbm.at[pl.ds(base_of(bb + 1), EBS)],
                            idxs[(I4 + 1) % 4], si[(I4 + 1) % 4])
                        pltpu.async_copy(
                            dst_hbm.at[pl.ds(base_of(bb + 1), EBS)],
                            idxd[(I4 + 1) % 4], si[(I4 + 1) % 4])

        for b in range(4 + 12 * m12, nblk):
            slot(b, steady=True)

        wait_gather(nblk - 1)
        issue_scatter(nblk - 1)
        wait_scatter(nblk - 3)
        wait_scatter(nblk - 2)
        wait_scatter(nblk - 1)

    @pl.when(cid == 0)
    def _():
        run_pipeline(W0, (sid * W0) * EBS)

    if W1 > 0:
        @pl.when(cid == 1)
        def _():
            run_pipeline(W1, (C1OFF + sid * W1) * EBS)

    plsc.subcore_barrier()
    for t in range(CHUNK // RC):
        r0 = sid * CHUNK + t * RC
        pltpu.sync_copy(acc_sh.at[pl.ds(r0, RC)], rows0.at[pl.ds(0, RC)])
        pltpu.sync_copy(rows0.at[pl.ds(0, RC)], out.at[cid, pl.ds(r0, RC)])


_RB = 2000
_G = (5,)


def _rows(w):
    return pl.BlockSpec((_RB, w), lambda i: (i, 0))


def _const(shape):
    return pl.BlockSpec(shape, lambda i: (0, 0))


def _dinv_body(p0, p1, o):
    deg = p0[:, 0:1] + p1[:, 0:1] - 1.0
    o[...] = lax.rsqrt(deg)


def _dinv_call(p0, p1):
    return pl.pallas_call(
        _dinv_body, grid=_G,
        in_specs=[_rows(D), _rows(D)],
        out_specs=_rows(1),
        out_shape=jax.ShapeDtypeStruct((N, 1), jnp.float32),
    )(p0, p1)


def _lin1_body(x, w, dinv, o):
    o[...] = dinv[...] * jnp.dot(x[...], w[...],
                                 preferred_element_type=jnp.float32)


def _lin1_call(x, W1, dinv):
    return pl.pallas_call(
        _lin1_body, grid=_G,
        in_specs=[_rows(D), _const((D, D)), _rows(1)],
        out_specs=_rows(D),
        out_shape=jax.ShapeDtypeStruct((N, D), jnp.float32),
    )(x, W1, dinv)


def _mid_body(p0, p1, g, dinv, b, w, o):
    h = jnp.maximum(dinv[...] * (p0[...] + p1[...] - g[...]) + b[...], 0.0)
    o[...] = dinv[...] * jnp.dot(h, w[...], preferred_element_type=jnp.float32)


def _mid_call(p0, p1, g, dinv, b, W):
    return pl.pallas_call(
        _mid_body, grid=_G,
        in_specs=[_rows(D), _rows(D), _rows(D), _rows(1),
                  _const((1, D)), _const((D, D))],
        out_specs=_rows(D),
        out_shape=jax.ShapeDtypeStruct((N, D), jnp.float32),
    )(p0, p1, g, dinv, b, W)


def _g3_body(p0, p1, g, dinv, b, o):
    h = jnp.maximum(dinv[...] * (p0[...] + p1[...] - g[...]) + b[...], 0.0)
    o[...] = dinv[...] * h


def _g3_call(p0, p1, g, dinv, b):
    return pl.pallas_call(
        _g3_body, grid=_G,
        in_specs=[_rows(D), _rows(D), _rows(D), _rows(1), _const((1, D))],
        out_specs=_rows(D),
        out_shape=jax.ShapeDtypeStruct((N, D), jnp.float32),
    )(p0, p1, g, dinv, b)


def _out_body(p0, p1, g, dinv, wmu, bmu, wls, bls, omu, ols):
    agg = dinv[...] * (p0[...] + p1[...] - g[...])
    omu[...] = jnp.dot(agg, wmu[...],
                       preferred_element_type=jnp.float32) + bmu[...]
    ols[...] = jnp.dot(agg, wls[...],
                       preferred_element_type=jnp.float32) + bls[...]


def _out_call(p0, p1, g, dinv, Wmu, bmu, Wls, bls):
    return pl.pallas_call(
        _out_body, grid=_G,
        in_specs=[_rows(D), _rows(D), _rows(D), _rows(1),
                  _const((D, DO)), _const((1, DO)),
                  _const((D, DO)), _const((1, DO))],
        out_specs=[_rows(DO), _rows(DO)],
        out_shape=[jax.ShapeDtypeStruct((N, DO), jnp.float32),
                   jax.ShapeDtypeStruct((N, DO), jnp.float32)],
    )(p0, p1, g, dinv, Wmu, bmu, Wls, bls)


def _pad_rows(g):
    return jnp.concatenate([g, jnp.zeros((NPAD - N, D), jnp.float32)], axis=0)


def kernel(x, edge_index, W1, b1, W2, b2, W_mu, b_mu, W_ls, b_ls):
    src = edge_index[0]
    dst = edge_index[1]
    dstp_deg = jnp.concatenate([dst, jnp.full((EPAD - E,), N, jnp.int32)])
    srcp = jnp.concatenate([src, jnp.zeros((SPAD - E,), jnp.int32)])
    dstp = jnp.concatenate([dst, jnp.full((SPAD - E,), N, jnp.int32)])

    onesp = jnp.ones((NPAD, D), jnp.float32)
    dd = _deg_kernel(dstp_deg, onesp)
    dinv = _dinv_call(dd[0], dd[1])

    b1r = b1.reshape(1, D)
    b2r = b2.reshape(1, D)
    bmur = b_mu.reshape(1, DO)
    blsr = b_ls.reshape(1, DO)

    g1 = _lin1_call(x, W1, dinv)
    p = _scatter_kernel(_pad_rows(g1), srcp, dstp)
    g2 = _mid_call(p[0], p[1], g1, dinv, b1r, W2)
    p = _scatter_kernel(_pad_rows(g2), srcp, dstp)
    g3 = _g3_call(p[0], p[1], g2, dinv, b2r)
    p = _scatter_kernel(_pad_rows(g3), srcp, dstp)
    mu, ls = _out_call(p[0], p[1], g3, dinv, W_mu, bmur, W_ls, blsr)
    return (mu, ls)


# speedup vs baseline: 1.8555x; 1.0836x over previous
import functools

import jax
import jax.numpy as jnp
from jax import lax
from jax.experimental import pallas as pl
from jax.experimental.pallas import tpu as pltpu
from jax.experimental.pallas import tpu_sc as plsc

N = 10000
E = 320000
D = 128
DO = 64

NW = 32
EPW = 10240
EB = 128
NBLK = EPW // EB
EPAD = NW * EPW
NPAD = 10240
CHUNK = NPAD // 16

_mesh = plsc.VectorSubcoreMesh(core_axis_name="c", subcore_axis_name="s")


NQ = (NBLK - 4) // 4

EBS = 112
W0 = 140
W1 = 39
WT = W0 + W1
C1OFF = 16 * W0
SPAD = 16 * WT * EBS + 2 * EBS
assert SPAD - 2 * EBS >= E


@functools.partial(
    pl.kernel,
    mesh=_mesh,
    out_type=jax.ShapeDtypeStruct((2, NPAD, D), jnp.float32),
    scratch_types=[
        pltpu.VMEM((EB, D), jnp.float32),
        pltpu.VMEM((EB,), jnp.int32),
        pltpu.VMEM((EB,), jnp.int32),
        pltpu.VMEM((EB,), jnp.int32),
        pltpu.VMEM((EB,), jnp.int32),
        pltpu.VMEM_SHARED((NPAD, D), jnp.float32),
        pltpu.SemaphoreType.DMA,
        pltpu.SemaphoreType.DMA,
        pltpu.SemaphoreType.DMA,
        pltpu.SemaphoreType.DMA,
        pltpu.SemaphoreType.DMA,
        pltpu.SemaphoreType.DMA,
    ],
)
def _deg_kernel(dst_hbm, ones_hbm, out, rows_v,
                idxd0, idxd1, idxd2, idxd3, acc_sh,
                ss0, ss1, si0, si1, si2, si3):
    cid = lax.axis_index("c")
    sid = lax.axis_index("s")
    wid = sid * 2 + cid
    idxd = [idxd0, idxd1, idxd2, idxd3]
    ss = [ss0, ss1]
    si = [si0, si1, si2, si3]

    for t in range(CHUNK // EB):
        r0 = sid * CHUNK + t * EB
        pltpu.sync_copy(ones_hbm.at[pl.ds(r0, EB)], rows_v)
        pltpu.sync_copy(rows_v, acc_sh.at[pl.ds(r0, EB)])
    plsc.subcore_barrier()

    def base_of(b):
        return wid * EPW + b * EB

    def prefetch(b, I):
        pltpu.async_copy(dst_hbm.at[pl.ds(base_of(b), EB)], idxd[I], si[I])

    def wait_prefetch(b, I):
        pltpu.make_async_copy(
            dst_hbm.at[pl.ds(base_of(b), EB)], idxd[I], si[I]).wait()

    def issue_scatter(B, I):
        pltpu.async_copy(rows_v, acc_sh.at[idxd[I]], ss[B], add=True)

    def wait_scatter(B, I):
        pltpu.make_async_copy(rows_v, acc_sh.at[idxd[I]], ss[B]).wait()

    for j in (0, 1):
        pltpu.sync_copy(dst_hbm.at[pl.ds(base_of(j), EB)], idxd[j])
        issue_scatter(j, j)
        prefetch(j + 2, j + 2)

    @pl.loop(0, NQ)
    def _(q):
        b0 = 2 + 4 * q
        for j in range(4):
            b = b0 + j
            B = j % 2
            I = (2 + j) % 4
            wait_scatter(B, (I + 2) % 4)
            wait_prefetch(b, I)
            issue_scatter(B, I)
            prefetch(b + 2, (I + 2) % 4)

    for j in (0, 1):
        b = NBLK - 2 + j
        B = j
        I = (2 + j) % 4
        wait_scatter(B, (I + 2) % 4)
        wait_prefetch(b, I)
        issue_scatter(B, I)
    wait_scatter(0, 2)
    wait_scatter(1, 3)

    plsc.subcore_barrier()
    for t in range(CHUNK // EB):
        r0 = sid * CHUNK + t * EB
        pltpu.sync_copy(acc_sh.at[pl.ds(r0, EB)], rows_v)
        pltpu.sync_copy(rows_v, out.at[cid, pl.ds(r0, EB)])


@functools.partial(
    pl.kernel,
    mesh=_mesh,
    out_type=jax.ShapeDtypeStruct((2, NPAD, D), jnp.float32),
    scratch_types=[
        pltpu.VMEM((EBS, D), jnp.float32),
        pltpu.VMEM((EBS, D), jnp.float32),
        pltpu.VMEM((EBS, D), jnp.float32),
        pltpu.VMEM((EBS,), jnp.int32),
        pltpu.VMEM((EBS,), jnp.int32),
        pltpu.VMEM((EBS,), jnp.int32),
        pltpu.VMEM((EBS,), jnp.int32),
        pltpu.VMEM((EBS,), jnp.int32),
        pltpu.VMEM((EBS,), jnp.int32),
        pltpu.VMEM((EBS,), jnp.int32),
        pltpu.VMEM((EBS,), jnp.int32),
        pltpu.VMEM_SHARED((NPAD, D), jnp.float32),
        pltpu.SemaphoreType.DMA,
        pltpu.SemaphoreType.DMA,
        pltpu.SemaphoreType.DMA,
        pltpu.SemaphoreType.DMA,
        pltpu.SemaphoreType.DMA,
        pltpu.SemaphoreType.DMA,
        pltpu.SemaphoreType.DMA,
        pltpu.SemaphoreType.DMA,
        pltpu.SemaphoreType.DMA,
        pltpu.SemaphoreType.DMA,
    ],
)
def _scatter_kernel(g_hbm, src_hbm, dst_hbm, out,
                    rows0, rows1, rows2,
                    idxs0, idxs1, idxs2, idxs3,
                    idxd0, idxd1, idxd2, idxd3,
                    acc_sh,
                    sg0, sg1, sg2, ss0, ss1, ss2, si0, si1, si2, si3):
    cid = lax.axis_index("c")
    sid = lax.axis_index("s")
    rows = [rows0, rows1, rows2]
    idxs = [idxs0, idxs1, idxs2, idxs3]
    idxd = [idxd0, idxd1, idxd2, idxd3]
    sg = [sg0, sg1, sg2]
    ss = [ss0, ss1, ss2]
    si = [si0, si1, si2, si3]

    RC = 80
    for t in range(CHUNK // RC):
        r0 = sid * CHUNK + t * RC
        pltpu.sync_copy(g_hbm.at[pl.ds(r0, RC)], rows0.at[pl.ds(0, RC)])
        pltpu.sync_copy(rows0.at[pl.ds(0, RC)], acc_sh.at[pl.ds(r0, RC)])
    plsc.subcore_barrier()

    def run_pipeline(nblk, tbase):
        def base_of(b):
            return tbase + b * EBS

        def prefetch(b):
            I = b % 4
            pltpu.async_copy(src_hbm.at[pl.ds(base_of(b), EBS)], idxs[I], si[I])
            pltpu.async_copy(dst_hbm.at[pl.ds(base_of(b), EBS)], idxd[I], si[I])

        def wait_prefetch(b):
            I = b % 4
            pltpu.make_async_copy(
                src_hbm.at[pl.ds(base_of(b), EBS)], idxs[I], si[I]).wait()
            pltpu.make_async_copy(
                dst_hbm.at[pl.ds(base_of(b), EBS)], idxd[I], si[I]).wait()

        def issue_gather(b):
            pltpu.async_copy(g_hbm.at[idxs[b % 4]], rows[b % 3], sg[b % 3])

        def wait_gather(b):
            pltpu.make_async_copy(
                g_hbm.at[idxs[b % 4]], rows[b % 3], sg[b % 3]).wait()

        def issue_scatter(b):
            pltpu.async_copy(rows[b % 3], acc_sh.at[idxd[b % 4]],
                             ss[b % 3], add=True)

        def wait_scatter(b):
            pltpu.make_async_copy(rows[b % 3], acc_sh.at[idxd[b % 4]],
                                  ss[b % 3]).wait()

        def slot(b, steady):
            if steady:
                wait_scatter(b - 3)
            wait_prefetch(b)
            issue_gather(b)
            wait_gather(b - 1)
            issue_scatter(b - 1)
            if b + 1 < nblk:
                prefetch(b + 1)

        pltpu.sync_copy(src_hbm.at[pl.ds(base_of(0), EBS)], idxs[0])
        pltpu.sync_copy(dst_hbm.at[pl.ds(base_of(0), EBS)], idxd[0])
        issue_gather(0)
        prefetch(1)
        for b in (1, 2):
            slot(b, steady=False)
        slot(3, steady=True)

        m12 = (nblk - 4) // 12
        if m12 > 0 and (nblk - 4) % 12 == 0:
            m12 -= 1

        if m12 > 0:
            @pl.loop(0, m12)
            def _(q):
                b0 = 4 + 12 * q
                for j in range(12):
                    bb = b0 + j
                    if True:
                        I3 = (4 + j) % 3
                        I4 = (4 + j) % 4
                        pltpu.make_async_copy(
                            rows[I3],
                            acc_sh.at[idxd[(I4 + 1) % 4]],
                            ss[I3]).wait()
                        pltpu.make_async_copy(
                            src_hbm.at[pl.ds(base_of(bb), EBS)],
                            idxs[I4], si[I4]).wait()
                        pltpu.make_async_copy(
                            dst_hbm.at[pl.ds(base_of(bb), EBS)],
                            idxd[I4], si[I4]).wait()
                        pltpu.async_copy(g_hbm.at[idxs[I4]],
                                         rows[I3], sg[I3])
                        pltpu.make_async_copy(
                            g_hbm.at[idxs[(I4 + 3) % 4]],
                            rows[(I3 + 2) % 3],
                            sg[(I3 + 2) % 3]).wait()
                        pltpu.async_copy(rows[(I3 + 2) % 3],
                                         acc_sh.at[idxd[(I4 + 3) % 4]],
                                         ss[(I3 + 2) % 3], add=True)
                        pltpu.async_copy(
                            src_hbm.at[pl.ds(base_of(bb + 1), EBS)],
                            idxs[(I4 + 1) % 4], si[(I4 + 1) % 4])
                        pltpu.async_copy(
                            dst_hbm.at[pl.ds(base_of(bb + 1), EBS)],
                            idxd[(I4 + 1) % 4], si[(I4 + 1) % 4])

        for b in range(4 + 12 * m12, nblk):
            slot(b, steady=True)

        wait_gather(nblk - 1)
        issue_scatter(nblk - 1)
        wait_scatter(nblk - 3)
        wait_scatter(nblk - 2)
        wait_scatter(nblk - 1)

    @pl.when(cid == 0)
    def _():
        run_pipeline(W0, (sid * W0) * EBS)

    if W1 > 0:
        @pl.when(cid == 1)
        def _():
            run_pipeline(W1, (C1OFF + sid * W1) * EBS)

    plsc.subcore_barrier()
    for t in range(CHUNK // RC):
        r0 = sid * CHUNK + t * RC
        pltpu.sync_copy(acc_sh.at[pl.ds(r0, RC)], rows0.at[pl.ds(0, RC)])
        pltpu.sync_copy(rows0.at[pl.ds(0, RC)], out.at[cid, pl.ds(r0, RC)])


_RB = 2000
_G = (5,)


def _rows(w):
    return pl.BlockSpec((_RB, w), lambda i: (i, 0))


def _const(shape):
    return pl.BlockSpec(shape, lambda i: (0, 0))


def _dinv_body(p0, p1, o):
    deg = p0[:, 0:1] + p1[:, 0:1] - 1.0
    o[...] = lax.rsqrt(deg)


def _dinv_call(p0, p1):
    return pl.pallas_call(
        _dinv_body, grid=_G,
        in_specs=[_rows(D), _rows(D)],
        out_specs=_rows(1),
        out_shape=jax.ShapeDtypeStruct((N, 1), jnp.float32),
    )(p0, p1)


def _lin1_body(x, w, dinv, o):
    o[...] = dinv[...] * jnp.dot(x[...], w[...],
                                 preferred_element_type=jnp.float32)


def _lin1_call(x, W1, dinv):
    return pl.pallas_call(
        _lin1_body, grid=_G,
        in_specs=[_rows(D), _const((D, D)), _rows(1)],
        out_specs=_rows(D),
        out_shape=jax.ShapeDtypeStruct((N, D), jnp.float32),
    )(x, W1, dinv)


def _mid_body(p0, p1, g, dinv, b, w, o):
    h = jnp.maximum(dinv[...] * (p0[...] + p1[...] - g[...]) + b[...], 0.0)
    o[...] = dinv[...] * jnp.dot(h, w[...], preferred_element_type=jnp.float32)


def _mid_call(p0, p1, g, dinv, b, W):
    return pl.pallas_call(
        _mid_body, grid=_G,
        in_specs=[_rows(D), _rows(D), _rows(D), _rows(1),
                  _const((1, D)), _const((D, D))],
        out_specs=_rows(D),
        out_shape=jax.ShapeDtypeStruct((N, D), jnp.float32),
    )(p0, p1, g, dinv, b, W)


def _g3_body(p0, p1, g, dinv, b, o):
    h = jnp.maximum(dinv[...] * (p0[...] + p1[...] - g[...]) + b[...], 0.0)
    o[...] = dinv[...] * h


def _g3_call(p0, p1, g, dinv, b):
    return pl.pallas_call(
        _g3_body, grid=_G,
        in_specs=[_rows(D), _rows(D), _rows(D), _rows(1), _const((1, D))],
        out_specs=_rows(D),
        out_shape=jax.ShapeDtypeStruct((N, D), jnp.float32),
    )(p0, p1, g, dinv, b)


def _out_body(p0, p1, g, dinv, wmu, bmu, wls, bls, omu, ols):
    agg = dinv[...] * (p0[...] + p1[...] - g[...])
    omu[...] = jnp.dot(agg, wmu[...],
                       preferred_element_type=jnp.float32) + bmu[...]
    ols[...] = jnp.dot(agg, wls[...],
                       preferred_element_type=jnp.float32) + bls[...]


def _out_call(p0, p1, g, dinv, Wmu, bmu, Wls, bls):
    return pl.pallas_call(
        _out_body, grid=_G,
        in_specs=[_rows(D), _rows(D), _rows(D), _rows(1),
                  _const((D, DO)), _const((1, DO)),
                  _const((D, DO)), _const((1, DO))],
        out_specs=[_rows(DO), _rows(DO)],
        out_shape=[jax.ShapeDtypeStruct((N, DO), jnp.float32),
                   jax.ShapeDtypeStruct((N, DO), jnp.float32)],
    )(p0, p1, g, dinv, Wmu, bmu, Wls, bls)


def _pad_rows(g):
    return jnp.concatenate([g, jnp.zeros((NPAD - N, D), jnp.float32)], axis=0)


def kernel(x, edge_index, W1, b1, W2, b2, W_mu, b_mu, W_ls, b_ls):
    src = edge_index[0]
    dst = edge_index[1]
    dstp_deg = jnp.concatenate([dst, jnp.full((EPAD - E,), N, jnp.int32)])
    srcp = jnp.concatenate([src, jnp.zeros((SPAD - E,), jnp.int32)])
    dstp = jnp.concatenate([dst, jnp.full((SPAD - E,), N, jnp.int32)])

    onesp = jnp.ones((NPAD, D), jnp.float32)
    dd = _deg_kernel(dstp_deg, onesp)
    dinv = _dinv_call(dd[0], dd[1])

    b1r = b1.reshape(1, D)
    b2r = b2.reshape(1, D)
    bmur = b_mu.reshape(1, DO)
    blsr = b_ls.reshape(1, DO)

    g1 = _lin1_call(x, W1, dinv)
    p = _scatter_kernel(_pad_rows(g1), srcp, dstp)
    g2 = _mid_call(p[0], p[1], g1, dinv, b1r, W2)
    p = _scatter_kernel(_pad_rows(g2), srcp, dstp)
    g3 = _g3_call(p[0], p[1], g2, dinv, b2r)
    p = _scatter_kernel(_pad_rows(g3), srcp, dstp)
    mu, ls = _out_call(p[0], p[1], g3, dinv, W_mu, bmur, W_ls, blsr)
    return (mu, ls)


# speedup vs baseline: 1.9183x; 1.0339x over previous
import functools

import jax
import jax.numpy as jnp
from jax import lax
from jax.experimental import pallas as pl
from jax.experimental.pallas import tpu as pltpu
from jax.experimental.pallas import tpu_sc as plsc

N = 10000
E = 320000
D = 128
DO = 64

NW = 32
EPW = 10240
EB = 128
NBLK = EPW // EB
EPAD = NW * EPW
NPAD = 10240
CHUNK = NPAD // 16

_mesh = plsc.VectorSubcoreMesh(core_axis_name="c", subcore_axis_name="s")


NQ = (NBLK - 4) // 4

EBS = 112
W0 = 132
W1 = 47
WT = W0 + W1
C1OFF = 16 * W0
SPAD = 16 * WT * EBS + 2 * EBS
assert SPAD - 2 * EBS >= E


@functools.partial(
    pl.kernel,
    mesh=_mesh,
    out_type=jax.ShapeDtypeStruct((2, NPAD, D), jnp.float32),
    scratch_types=[
        pltpu.VMEM((EB, D), jnp.float32),
        pltpu.VMEM((EB,), jnp.int32),
        pltpu.VMEM((EB,), jnp.int32),
        pltpu.VMEM((EB,), jnp.int32),
        pltpu.VMEM((EB,), jnp.int32),
        pltpu.VMEM_SHARED((NPAD, D), jnp.float32),
        pltpu.SemaphoreType.DMA,
        pltpu.SemaphoreType.DMA,
        pltpu.SemaphoreType.DMA,
        pltpu.SemaphoreType.DMA,
        pltpu.SemaphoreType.DMA,
        pltpu.SemaphoreType.DMA,
    ],
)
def _deg_kernel(dst_hbm, ones_hbm, out, rows_v,
                idxd0, idxd1, idxd2, idxd3, acc_sh,
                ss0, ss1, si0, si1, si2, si3):
    cid = lax.axis_index("c")
    sid = lax.axis_index("s")
    wid = sid * 2 + cid
    idxd = [idxd0, idxd1, idxd2, idxd3]
    ss = [ss0, ss1]
    si = [si0, si1, si2, si3]

    for t in range(CHUNK // EB):
        r0 = sid * CHUNK + t * EB
        pltpu.sync_copy(ones_hbm.at[pl.ds(r0, EB)], rows_v)
        pltpu.sync_copy(rows_v, acc_sh.at[pl.ds(r0, EB)])
    plsc.subcore_barrier()

    def base_of(b):
        return wid * EPW + b * EB

    def prefetch(b, I):
        pltpu.async_copy(dst_hbm.at[pl.ds(base_of(b), EB)], idxd[I], si[I])

    def wait_prefetch(b, I):
        pltpu.make_async_copy(
            dst_hbm.at[pl.ds(base_of(b), EB)], idxd[I], si[I]).wait()

    def issue_scatter(B, I):
        pltpu.async_copy(rows_v, acc_sh.at[idxd[I]], ss[B], add=True)

    def wait_scatter(B, I):
        pltpu.make_async_copy(rows_v, acc_sh.at[idxd[I]], ss[B]).wait()

    for j in (0, 1):
        pltpu.sync_copy(dst_hbm.at[pl.ds(base_of(j), EB)], idxd[j])
        issue_scatter(j, j)
        prefetch(j + 2, j + 2)

    @pl.loop(0, NQ)
    def _(q):
        b0 = 2 + 4 * q
        for j in range(4):
            b = b0 + j
            B = j % 2
            I = (2 + j) % 4
            wait_scatter(B, (I + 2) % 4)
            wait_prefetch(b, I)
            issue_scatter(B, I)
            prefetch(b + 2, (I + 2) % 4)

    for j in (0, 1):
        b = NBLK - 2 + j
        B = j
        I = (2 + j) % 4
        wait_scatter(B, (I + 2) % 4)
        wait_prefetch(b, I)
        issue_scatter(B, I)
    wait_scatter(0, 2)
    wait_scatter(1, 3)

    plsc.subcore_barrier()
    for t in range(CHUNK // EB):
        r0 = sid * CHUNK + t * EB
        pltpu.sync_copy(acc_sh.at[pl.ds(r0, EB)], rows_v)
        pltpu.sync_copy(rows_v, out.at[cid, pl.ds(r0, EB)])


@functools.partial(
    pl.kernel,
    mesh=_mesh,
    out_type=jax.ShapeDtypeStruct((2, NPAD, D), jnp.float32),
    scratch_types=[
        pltpu.VMEM((EBS, D), jnp.float32),
        pltpu.VMEM((EBS, D), jnp.float32),
        pltpu.VMEM((EBS, D), jnp.float32),
        pltpu.VMEM((EBS,), jnp.int32),
        pltpu.VMEM((EBS,), jnp.int32),
        pltpu.VMEM((EBS,), jnp.int32),
        pltpu.VMEM((EBS,), jnp.int32),
        pltpu.VMEM((EBS,), jnp.int32),
        pltpu.VMEM((EBS,), jnp.int32),
        pltpu.VMEM((EBS,), jnp.int32),
        pltpu.VMEM((EBS,), jnp.int32),
        pltpu.VMEM_SHARED((NPAD, D), jnp.float32),
        pltpu.SemaphoreType.DMA,
        pltpu.SemaphoreType.DMA,
        pltpu.SemaphoreType.DMA,
        pltpu.SemaphoreType.DMA,
        pltpu.SemaphoreType.DMA,
        pltpu.SemaphoreType.DMA,
        pltpu.SemaphoreType.DMA,
        pltpu.SemaphoreType.DMA,
        pltpu.SemaphoreType.DMA,
        pltpu.SemaphoreType.DMA,
    ],
)
def _scatter_kernel(g_hbm, src_hbm, dst_hbm, out,
                    rows0, rows1, rows2,
                    idxs0, idxs1, idxs2, idxs3,
                    idxd0, idxd1, idxd2, idxd3,
                    acc_sh,
                    sg0, sg1, sg2, ss0, ss1, ss2, si0, si1, si2, si3):
    cid = lax.axis_index("c")
    sid = lax.axis_index("s")
    rows = [rows0, rows1, rows2]
    idxs = [idxs0, idxs1, idxs2, idxs3]
    idxd = [idxd0, idxd1, idxd2, idxd3]
    sg = [sg0, sg1, sg2]
    ss = [ss0, ss1, ss2]
    si = [si0, si1, si2, si3]

    RC = 80
    for t in range(CHUNK // RC):
        r0 = sid * CHUNK + t * RC
        pltpu.sync_copy(g_hbm.at[pl.ds(r0, RC)], rows0.at[pl.ds(0, RC)])
        pltpu.sync_copy(rows0.at[pl.ds(0, RC)], acc_sh.at[pl.ds(r0, RC)])
    plsc.subcore_barrier()

    def run_pipeline(nblk, tbase):
        def base_of(b):
            return tbase + b * EBS

        def prefetch(b):
            I = b % 4
            pltpu.async_copy(src_hbm.at[pl.ds(base_of(b), EBS)], idxs[I], si[I])
            pltpu.async_copy(dst_hbm.at[pl.ds(base_of(b), EBS)], idxd[I], si[I])

        def wait_prefetch(b):
            I = b % 4
            pltpu.make_async_copy(
                src_hbm.at[pl.ds(base_of(b), EBS)], idxs[I], si[I]).wait()
            pltpu.make_async_copy(
                dst_hbm.at[pl.ds(base_of(b), EBS)], idxd[I], si[I]).wait()

        def issue_gather(b):
            pltpu.async_copy(g_hbm.at[idxs[b % 4]], rows[b % 3], sg[b % 3])

        def wait_gather(b):
            pltpu.make_async_copy(
                g_hbm.at[idxs[b % 4]], rows[b % 3], sg[b % 3]).wait()

        def issue_scatter(b):
            pltpu.async_copy(rows[b % 3], acc_sh.at[idxd[b % 4]],
                             ss[b % 3], add=True)

        def wait_scatter(b):
            pltpu.make_async_copy(rows[b % 3], acc_sh.at[idxd[b % 4]],
                                  ss[b % 3]).wait()

        def slot(b, steady):
            if steady:
                wait_scatter(b - 3)
            wait_prefetch(b)
            issue_gather(b)
            wait_gather(b - 1)
            issue_scatter(b - 1)
            if b + 1 < nblk:
                prefetch(b + 1)

        pltpu.sync_copy(src_hbm.at[pl.ds(base_of(0), EBS)], idxs[0])
        pltpu.sync_copy(dst_hbm.at[pl.ds(base_of(0), EBS)], idxd[0])
        issue_gather(0)
        prefetch(1)
        for b in (1, 2):
            slot(b, steady=False)
        slot(3, steady=True)

        m12 = (nblk - 4) // 12
        if m12 > 0 and (nblk - 4) % 12 == 0:
            m12 -= 1

        if m12 > 0:
            @pl.loop(0, m12)
            def _(q):
                b0 = 4 + 12 * q
                for j in range(12):
                    bb = b0 + j
                    if True:
                        I3 = (4 + j) % 3
                        I4 = (4 + j) % 4
                        pltpu.make_async_copy(
                            rows[I3],
                            acc_sh.at[idxd[(I4 + 1) % 4]],
                            ss[I3]).wait()
                        pltpu.make_async_copy(
                            src_hbm.at[pl.ds(base_of(bb), EBS)],
                            idxs[I4], si[I4]).wait()
                        pltpu.make_async_copy(
                            dst_hbm.at[pl.ds(base_of(bb), EBS)],
                            idxd[I4], si[I4]).wait()
                        pltpu.async_copy(g_hbm.at[idxs[I4]],
                                         rows[I3], sg[I3])
                        pltpu.make_async_copy(
                            g_hbm.at[idxs[(I4 + 3) % 4]],
                            rows[(I3 + 2) % 3],
                            sg[(I3 + 2) % 3]).wait()
                        pltpu.async_copy(rows[(I3 + 2) % 3],
                                         acc_sh.at[idxd[(I4 + 3) % 4]],
                                         ss[(I3 + 2) % 3], add=True)
                        pltpu.async_copy(
                            src_hbm.at[pl.ds(base_of(bb + 1), EBS)],
                            idxs[(I4 + 1) % 4], si[(I4 + 1) % 4])
                        pltpu.async_copy(
                            dst_hbm.at[pl.ds(base_of(bb + 1), EBS)],
                            idxd[(I4 + 1) % 4], si[(I4 + 1) % 4])

        for b in range(4 + 12 * m12, nblk):
            slot(b, steady=True)

        wait_gather(nblk - 1)
        issue_scatter(nblk - 1)
        wait_scatter(nblk - 3)
        wait_scatter(nblk - 2)
        wait_scatter(nblk - 1)

    @pl.when(cid == 0)
    def _():
        run_pipeline(W0, (sid * W0) * EBS)

    if W1 > 0:
        @pl.when(cid == 1)
        def _():
            run_pipeline(W1, (C1OFF + sid * W1) * EBS)

    plsc.subcore_barrier()
    for t in range(CHUNK // RC):
        r0 = sid * CHUNK + t * RC
        pltpu.sync_copy(acc_sh.at[pl.ds(r0, RC)], rows0.at[pl.ds(0, RC)])
        pltpu.sync_copy(rows0.at[pl.ds(0, RC)], out.at[cid, pl.ds(r0, RC)])


_RB = 2000
_G = (5,)


def _rows(w):
    return pl.BlockSpec((_RB, w), lambda i: (i, 0))


def _const(shape):
    return pl.BlockSpec(shape, lambda i: (0, 0))


def _dinv_body(p0, p1, o):
    deg = p0[:, 0:1] + p1[:, 0:1] - 1.0
    o[...] = lax.rsqrt(deg)


def _dinv_call(p0, p1):
    return pl.pallas_call(
        _dinv_body, grid=_G,
        in_specs=[_rows(D), _rows(D)],
        out_specs=_rows(1),
        out_shape=jax.ShapeDtypeStruct((N, 1), jnp.float32),
    )(p0, p1)


def _lin1_body(x, w, dinv, o):
    o[...] = dinv[...] * jnp.dot(x[...], w[...],
                                 preferred_element_type=jnp.float32)


def _lin1_call(x, W1, dinv):
    return pl.pallas_call(
        _lin1_body, grid=_G,
        in_specs=[_rows(D), _const((D, D)), _rows(1)],
        out_specs=_rows(D),
        out_shape=jax.ShapeDtypeStruct((N, D), jnp.float32),
    )(x, W1, dinv)


def _mid_body(p0, p1, g, dinv, b, w, o):
    h = jnp.maximum(dinv[...] * (p0[...] + p1[...] - g[...]) + b[...], 0.0)
    o[...] = dinv[...] * jnp.dot(h, w[...], preferred_element_type=jnp.float32)


def _mid_call(p0, p1, g, dinv, b, W):
    return pl.pallas_call(
        _mid_body, grid=_G,
        in_specs=[_rows(D), _rows(D), _rows(D), _rows(1),
                  _const((1, D)), _const((D, D))],
        out_specs=_rows(D),
        out_shape=jax.ShapeDtypeStruct((N, D), jnp.float32),
    )(p0, p1, g, dinv, b, W)


def _g3_body(p0, p1, g, dinv, b, o):
    h = jnp.maximum(dinv[...] * (p0[...] + p1[...] - g[...]) + b[...], 0.0)
    o[...] = dinv[...] * h


def _g3_call(p0, p1, g, dinv, b):
    return pl.pallas_call(
        _g3_body, grid=_G,
        in_specs=[_rows(D), _rows(D), _rows(D), _rows(1), _const((1, D))],
        out_specs=_rows(D),
        out_shape=jax.ShapeDtypeStruct((N, D), jnp.float32),
    )(p0, p1, g, dinv, b)


def _out_body(p0, p1, g, dinv, wmu, bmu, wls, bls, omu, ols):
    agg = dinv[...] * (p0[...] + p1[...] - g[...])
    omu[...] = jnp.dot(agg, wmu[...],
                       preferred_element_type=jnp.float32) + bmu[...]
    ols[...] = jnp.dot(agg, wls[...],
                       preferred_element_type=jnp.float32) + bls[...]


def _out_call(p0, p1, g, dinv, Wmu, bmu, Wls, bls):
    return pl.pallas_call(
        _out_body, grid=_G,
        in_specs=[_rows(D), _rows(D), _rows(D), _rows(1),
                  _const((D, DO)), _const((1, DO)),
                  _const((D, DO)), _const((1, DO))],
        out_specs=[_rows(DO), _rows(DO)],
        out_shape=[jax.ShapeDtypeStruct((N, DO), jnp.float32),
                   jax.ShapeDtypeStruct((N, DO), jnp.float32)],
    )(p0, p1, g, dinv, Wmu, bmu, Wls, bls)


def _pad_rows(g):
    return jnp.concatenate([g, jnp.zeros((NPAD - N, D), jnp.float32)], axis=0)


def kernel(x, edge_index, W1, b1, W2, b2, W_mu, b_mu, W_ls, b_ls):
    src = edge_index[0]
    dst = edge_index[1]
    dstp_deg = jnp.concatenate([dst, jnp.full((EPAD - E,), N, jnp.int32)])
    srcp = jnp.concatenate([src, jnp.zeros((SPAD - E,), jnp.int32)])
    dstp = jnp.concatenate([dst, jnp.full((SPAD - E,), N, jnp.int32)])

    onesp = jnp.ones((NPAD, D), jnp.float32)
    dd = _deg_kernel(dstp_deg, onesp)
    dinv = _dinv_call(dd[0], dd[1])

    b1r = b1.reshape(1, D)
    b2r = b2.reshape(1, D)
    bmur = b_mu.reshape(1, DO)
    blsr = b_ls.reshape(1, DO)

    g1 = _lin1_call(x, W1, dinv)
    p = _scatter_kernel(_pad_rows(g1), srcp, dstp)
    g2 = _mid_call(p[0], p[1], g1, dinv, b1r, W2)
    p = _scatter_kernel(_pad_rows(g2), srcp, dstp)
    g3 = _g3_call(p[0], p[1], g2, dinv, b2r)
    p = _scatter_kernel(_pad_rows(g3), srcp, dstp)
    mu, ls = _out_call(p[0], p[1], g3, dinv, W_mu, bmur, W_ls, blsr)
    return (mu, ls)


# speedup vs baseline: 1.9834x; 1.0339x over previous
import functools

import jax
import jax.numpy as jnp
from jax import lax
from jax.experimental import pallas as pl
from jax.experimental.pallas import tpu as pltpu
from jax.experimental.pallas import tpu_sc as plsc

N = 10000
E = 320000
D = 128
DO = 64

NW = 32
EPW = 10240
EB = 128
NBLK = EPW // EB
EPAD = NW * EPW
NPAD = 10240
CHUNK = NPAD // 16

_mesh = plsc.VectorSubcoreMesh(core_axis_name="c", subcore_axis_name="s")


NQ = (NBLK - 4) // 4

EBS = 112
W0 = 124
W1 = 55
WT = W0 + W1
C1OFF = 16 * W0
SPAD = 16 * WT * EBS + 2 * EBS
assert SPAD - 2 * EBS >= E


@functools.partial(
    pl.kernel,
    mesh=_mesh,
    out_type=jax.ShapeDtypeStruct((2, NPAD, D), jnp.float32),
    scratch_types=[
        pltpu.VMEM((EB, D), jnp.float32),
        pltpu.VMEM((EB,), jnp.int32),
        pltpu.VMEM((EB,), jnp.int32),
        pltpu.VMEM((EB,), jnp.int32),
        pltpu.VMEM((EB,), jnp.int32),
        pltpu.VMEM_SHARED((NPAD, D), jnp.float32),
        pltpu.SemaphoreType.DMA,
        pltpu.SemaphoreType.DMA,
        pltpu.SemaphoreType.DMA,
        pltpu.SemaphoreType.DMA,
        pltpu.SemaphoreType.DMA,
        pltpu.SemaphoreType.DMA,
    ],
)
def _deg_kernel(dst_hbm, ones_hbm, out, rows_v,
                idxd0, idxd1, idxd2, idxd3, acc_sh,
                ss0, ss1, si0, si1, si2, si3):
    cid = lax.axis_index("c")
    sid = lax.axis_index("s")
    wid = sid * 2 + cid
    idxd = [idxd0, idxd1, idxd2, idxd3]
    ss = [ss0, ss1]
    si = [si0, si1, si2, si3]

    for t in range(CHUNK // EB):
        r0 = sid * CHUNK + t * EB
        pltpu.sync_copy(ones_hbm.at[pl.ds(r0, EB)], rows_v)
        pltpu.sync_copy(rows_v, acc_sh.at[pl.ds(r0, EB)])
    plsc.subcore_barrier()

    def base_of(b):
        return wid * EPW + b * EB

    def prefetch(b, I):
        pltpu.async_copy(dst_hbm.at[pl.ds(base_of(b), EB)], idxd[I], si[I])

    def wait_prefetch(b, I):
        pltpu.make_async_copy(
            dst_hbm.at[pl.ds(base_of(b), EB)], idxd[I], si[I]).wait()

    def issue_scatter(B, I):
        pltpu.async_copy(rows_v, acc_sh.at[idxd[I]], ss[B], add=True)

    def wait_scatter(B, I):
        pltpu.make_async_copy(rows_v, acc_sh.at[idxd[I]], ss[B]).wait()

    for j in (0, 1):
        pltpu.sync_copy(dst_hbm.at[pl.ds(base_of(j), EB)], idxd[j])
        issue_scatter(j, j)
        prefetch(j + 2, j + 2)

    @pl.loop(0, NQ)
    def _(q):
        b0 = 2 + 4 * q
        for j in range(4):
            b = b0 + j
            B = j % 2
            I = (2 + j) % 4
            wait_scatter(B, (I + 2) % 4)
            wait_prefetch(b, I)
            issue_scatter(B, I)
            prefetch(b + 2, (I + 2) % 4)

    for j in (0, 1):
        b = NBLK - 2 + j
        B = j
        I = (2 + j) % 4
        wait_scatter(B, (I + 2) % 4)
        wait_prefetch(b, I)
        issue_scatter(B, I)
    wait_scatter(0, 2)
    wait_scatter(1, 3)

    plsc.subcore_barrier()
    for t in range(CHUNK // EB):
        r0 = sid * CHUNK + t * EB
        pltpu.sync_copy(acc_sh.at[pl.ds(r0, EB)], rows_v)
        pltpu.sync_copy(rows_v, out.at[cid, pl.ds(r0, EB)])


@functools.partial(
    pl.kernel,
    mesh=_mesh,
    out_type=jax.ShapeDtypeStruct((2, NPAD, D), jnp.float32),
    scratch_types=[
        pltpu.VMEM((EBS, D), jnp.float32),
        pltpu.VMEM((EBS, D), jnp.float32),
        pltpu.VMEM((EBS, D), jnp.float32),
        pltpu.VMEM((EBS,), jnp.int32),
        pltpu.VMEM((EBS,), jnp.int32),
        pltpu.VMEM((EBS,), jnp.int32),
        pltpu.VMEM((EBS,), jnp.int32),
        pltpu.VMEM((EBS,), jnp.int32),
        pltpu.VMEM((EBS,), jnp.int32),
        pltpu.VMEM((EBS,), jnp.int32),
        pltpu.VMEM((EBS,), jnp.int32),
        pltpu.VMEM_SHARED((NPAD, D), jnp.float32),
        pltpu.SemaphoreType.DMA,
        pltpu.SemaphoreType.DMA,
        pltpu.SemaphoreType.DMA,
        pltpu.SemaphoreType.DMA,
        pltpu.SemaphoreType.DMA,
        pltpu.SemaphoreType.DMA,
        pltpu.SemaphoreType.DMA,
        pltpu.SemaphoreType.DMA,
        pltpu.SemaphoreType.DMA,
        pltpu.SemaphoreType.DMA,
    ],
)
def _scatter_kernel(g_hbm, src_hbm, dst_hbm, out,
                    rows0, rows1, rows2,
                    idxs0, idxs1, idxs2, idxs3,
                    idxd0, idxd1, idxd2, idxd3,
                    acc_sh,
                    sg0, sg1, sg2, ss0, ss1, ss2, si0, si1, si2, si3):
    cid = lax.axis_index("c")
    sid = lax.axis_index("s")
    rows = [rows0, rows1, rows2]
    idxs = [idxs0, idxs1, idxs2, idxs3]
    idxd = [idxd0, idxd1, idxd2, idxd3]
    sg = [sg0, sg1, sg2]
    ss = [ss0, ss1, ss2]
    si = [si0, si1, si2, si3]

    RC = 80
    for t in range(CHUNK // RC):
        r0 = sid * CHUNK + t * RC
        pltpu.sync_copy(g_hbm.at[pl.ds(r0, RC)], rows0.at[pl.ds(0, RC)])
        pltpu.sync_copy(rows0.at[pl.ds(0, RC)], acc_sh.at[pl.ds(r0, RC)])
    plsc.subcore_barrier()

    def run_pipeline(nblk, tbase):
        def base_of(b):
            return tbase + b * EBS

        def prefetch(b):
            I = b % 4
            pltpu.async_copy(src_hbm.at[pl.ds(base_of(b), EBS)], idxs[I], si[I])
            pltpu.async_copy(dst_hbm.at[pl.ds(base_of(b), EBS)], idxd[I], si[I])

        def wait_prefetch(b):
            I = b % 4
            pltpu.make_async_copy(
                src_hbm.at[pl.ds(base_of(b), EBS)], idxs[I], si[I]).wait()
            pltpu.make_async_copy(
                dst_hbm.at[pl.ds(base_of(b), EBS)], idxd[I], si[I]).wait()

        def issue_gather(b):
            pltpu.async_copy(g_hbm.at[idxs[b % 4]], rows[b % 3], sg[b % 3])

        def wait_gather(b):
            pltpu.make_async_copy(
                g_hbm.at[idxs[b % 4]], rows[b % 3], sg[b % 3]).wait()

        def issue_scatter(b):
            pltpu.async_copy(rows[b % 3], acc_sh.at[idxd[b % 4]],
                             ss[b % 3], add=True)

        def wait_scatter(b):
            pltpu.make_async_copy(rows[b % 3], acc_sh.at[idxd[b % 4]],
                                  ss[b % 3]).wait()

        def slot(b, steady):
            if steady:
                wait_scatter(b - 3)
            wait_prefetch(b)
            issue_gather(b)
            wait_gather(b - 1)
            issue_scatter(b - 1)
            if b + 1 < nblk:
                prefetch(b + 1)

        pltpu.sync_copy(src_hbm.at[pl.ds(base_of(0), EBS)], idxs[0])
        pltpu.sync_copy(dst_hbm.at[pl.ds(base_of(0), EBS)], idxd[0])
        issue_gather(0)
        prefetch(1)
        for b in (1, 2):
            slot(b, steady=False)
        slot(3, steady=True)

        m12 = (nblk - 4) // 12
        if m12 > 0 and (nblk - 4) % 12 == 0:
            m12 -= 1

        if m12 > 0:
            @pl.loop(0, m12)
            def _(q):
                b0 = 4 + 12 * q
                for j in range(12):
                    bb = b0 + j
                    if True:
                        I3 = (4 + j) % 3
                        I4 = (4 + j) % 4
                        pltpu.make_async_copy(
                            rows[I3],
                            acc_sh.at[idxd[(I4 + 1) % 4]],
                            ss[I3]).wait()
                        pltpu.make_async_copy(
                            src_hbm.at[pl.ds(base_of(bb), EBS)],
                            idxs[I4], si[I4]).wait()
                        pltpu.make_async_copy(
                            dst_hbm.at[pl.ds(base_of(bb), EBS)],
                            idxd[I4], si[I4]).wait()
                        pltpu.async_copy(g_hbm.at[idxs[I4]],
                                         rows[I3], sg[I3])
                        pltpu.make_async_copy(
                            g_hbm.at[idxs[(I4 + 3) % 4]],
                            rows[(I3 + 2) % 3],
                            sg[(I3 + 2) % 3]).wait()
                        pltpu.async_copy(rows[(I3 + 2) % 3],
                                         acc_sh.at[idxd[(I4 + 3) % 4]],
                                         ss[(I3 + 2) % 3], add=True)
                        pltpu.async_copy(
                            src_hbm.at[pl.ds(base_of(bb + 1), EBS)],
                            idxs[(I4 + 1) % 4], si[(I4 + 1) % 4])
                        pltpu.async_copy(
                            dst_hbm.at[pl.ds(base_of(bb + 1), EBS)],
                            idxd[(I4 + 1) % 4], si[(I4 + 1) % 4])

        for b in range(4 + 12 * m12, nblk):
            slot(b, steady=True)

        wait_gather(nblk - 1)
        issue_scatter(nblk - 1)
        wait_scatter(nblk - 3)
        wait_scatter(nblk - 2)
        wait_scatter(nblk - 1)

    @pl.when(cid == 0)
    def _():
        run_pipeline(W0, (sid * W0) * EBS)

    if W1 > 0:
        @pl.when(cid == 1)
        def _():
            run_pipeline(W1, (C1OFF + sid * W1) * EBS)

    plsc.subcore_barrier()
    for t in range(CHUNK // RC):
        r0 = sid * CHUNK + t * RC
        pltpu.sync_copy(acc_sh.at[pl.ds(r0, RC)], rows0.at[pl.ds(0, RC)])
        pltpu.sync_copy(rows0.at[pl.ds(0, RC)], out.at[cid, pl.ds(r0, RC)])


_RB = 2000
_G = (5,)


def _rows(w):
    return pl.BlockSpec((_RB, w), lambda i: (i, 0))


def _const(shape):
    return pl.BlockSpec(shape, lambda i: (0, 0))


def _dinv_body(p0, p1, o):
    deg = p0[:, 0:1] + p1[:, 0:1] - 1.0
    o[...] = lax.rsqrt(deg)


def _dinv_call(p0, p1):
    return pl.pallas_call(
        _dinv_body, grid=_G,
        in_specs=[_rows(D), _rows(D)],
        out_specs=_rows(1),
        out_shape=jax.ShapeDtypeStruct((N, 1), jnp.float32),
    )(p0, p1)


def _lin1_body(x, w, dinv, o):
    o[...] = dinv[...] * jnp.dot(x[...], w[...],
                                 preferred_element_type=jnp.float32)


def _lin1_call(x, W1, dinv):
    return pl.pallas_call(
        _lin1_body, grid=_G,
        in_specs=[_rows(D), _const((D, D)), _rows(1)],
        out_specs=_rows(D),
        out_shape=jax.ShapeDtypeStruct((N, D), jnp.float32),
    )(x, W1, dinv)


def _mid_body(p0, p1, g, dinv, b, w, o):
    h = jnp.maximum(dinv[...] * (p0[...] + p1[...] - g[...]) + b[...], 0.0)
    o[...] = dinv[...] * jnp.dot(h, w[...], preferred_element_type=jnp.float32)


def _mid_call(p0, p1, g, dinv, b, W):
    return pl.pallas_call(
        _mid_body, grid=_G,
        in_specs=[_rows(D), _rows(D), _rows(D), _rows(1),
                  _const((1, D)), _const((D, D))],
        out_specs=_rows(D),
        out_shape=jax.ShapeDtypeStruct((N, D), jnp.float32),
    )(p0, p1, g, dinv, b, W)


def _g3_body(p0, p1, g, dinv, b, o):
    h = jnp.maximum(dinv[...] * (p0[...] + p1[...] - g[...]) + b[...], 0.0)
    o[...] = dinv[...] * h


def _g3_call(p0, p1, g, dinv, b):
    return pl.pallas_call(
        _g3_body, grid=_G,
        in_specs=[_rows(D), _rows(D), _rows(D), _rows(1), _const((1, D))],
        out_specs=_rows(D),
        out_shape=jax.ShapeDtypeStruct((N, D), jnp.float32),
    )(p0, p1, g, dinv, b)


def _out_body(p0, p1, g, dinv, wmu, bmu, wls, bls, omu, ols):
    agg = dinv[...] * (p0[...] + p1[...] - g[...])
    omu[...] = jnp.dot(agg, wmu[...],
                       preferred_element_type=jnp.float32) + bmu[...]
    ols[...] = jnp.dot(agg, wls[...],
                       preferred_element_type=jnp.float32) + bls[...]


def _out_call(p0, p1, g, dinv, Wmu, bmu, Wls, bls):
    return pl.pallas_call(
        _out_body, grid=_G,
        in_specs=[_rows(D), _rows(D), _rows(D), _rows(1),
                  _const((D, DO)), _const((1, DO)),
                  _const((D, DO)), _const((1, DO))],
        out_specs=[_rows(DO), _rows(DO)],
        out_shape=[jax.ShapeDtypeStruct((N, DO), jnp.float32),
                   jax.ShapeDtypeStruct((N, DO), jnp.float32)],
    )(p0, p1, g, dinv, Wmu, bmu, Wls, bls)


def _pad_rows(g):
    return jnp.concatenate([g, jnp.zeros((NPAD - N, D), jnp.float32)], axis=0)


def kernel(x, edge_index, W1, b1, W2, b2, W_mu, b_mu, W_ls, b_ls):
    src = edge_index[0]
    dst = edge_index[1]
    dstp_deg = jnp.concatenate([dst, jnp.full((EPAD - E,), N, jnp.int32)])
    srcp = jnp.concatenate([src, jnp.zeros((SPAD - E,), jnp.int32)])
    dstp = jnp.concatenate([dst, jnp.full((SPAD - E,), N, jnp.int32)])

    onesp = jnp.ones((NPAD, D), jnp.float32)
    dd = _deg_kernel(dstp_deg, onesp)
    dinv = _dinv_call(dd[0], dd[1])

    b1r = b1.reshape(1, D)
    b2r = b2.reshape(1, D)
    bmur = b_mu.reshape(1, DO)
    blsr = b_ls.reshape(1, DO)

    g1 = _lin1_call(x, W1, dinv)
    p = _scatter_kernel(_pad_rows(g1), srcp, dstp)
    g2 = _mid_call(p[0], p[1], g1, dinv, b1r, W2)
    p = _scatter_kernel(_pad_rows(g2), srcp, dstp)
    g3 = _g3_call(p[0], p[1], g2, dinv, b2r)
    p = _scatter_kernel(_pad_rows(g3), srcp, dstp)
    mu, ls = _out_call(p[0], p[1], g3, dinv, W_mu, bmur, W_ls, blsr)
    return (mu, ls)
